# Initial kernel scaffold; baseline (speedup 1.0000x reference)
import jax
import jax.numpy as jnp
from jax.experimental import pallas as pl


def kernel(x, edge_index, edge_attr, w0, w1):
    raise NotImplementedError("write your pallas kernel here")


# speedup vs baseline: 3.3763x; 3.3763x over previous
import functools
import jax
import jax.numpy as jnp
import numpy as np
from jax.experimental import pallas as pl

N_NODES = 50000
IN_CH = 1443
K1 = 25000
K2 = 12500
ROWS_BLK = 1000
N_BLOCKS = N_NODES // ROWS_BLK
PAD_N = 50176
PAD_R, PAD_C = 392, 128


def _matvec2_body(x_ref, w_ref, o_ref):
    o_ref[...] = jnp.dot(x_ref[...], w_ref[...],
                         preferred_element_type=jnp.float32)


def _ordered_key(f):
    b = jax.lax.bitcast_convert_type(f, jnp.int32)
    return b ^ ((b >> 31) & jnp.int32(0x7FFFFFFF))


def _count_gt_threshold(key, k):
    def body(_, lohi):
        lo, hi = lohi
        mid = (lo & hi) + ((lo ^ hi) >> 1)
        cnt = jnp.sum((key > mid).astype(jnp.int32))
        big = cnt >= k
        return (jnp.where(big, mid + 1, lo), jnp.where(big, hi, mid))
    lo = jnp.int32(-2**31)
    hi = jnp.int32(2**31 - 1)
    lo, hi = jax.lax.fori_loop(0, 32, body, (lo, hi))
    return lo


def _select_topk(key, idx, k):
    t = _count_gt_threshold(key, k)
    above = key > t
    m = k - jnp.sum(above.astype(jnp.int32))
    tie = key == t

    def body(_, lohi):
        lo, hi = lohi
        mid = (lo + hi) >> 1
        cnt = jnp.sum((tie & (idx < mid)).astype(jnp.int32))
        big = cnt >= m
        return (jnp.where(big, lo, mid + 1), jnp.where(big, mid, hi))
    lo, hi = jax.lax.fori_loop(0, 17, body,
                               (jnp.int32(0), jnp.int32(PAD_N + 1)))
    return above | (tie & (idx < lo))


def _coeff_body(d0_ref, d1_ref, w0_ref, w1_ref, o_ref):
    n0 = jnp.sqrt(jnp.sum(w0_ref[...] * w0_ref[...])) + 1e-16
    n1 = jnp.sqrt(jnp.sum(w1_ref[...] * w1_ref[...])) + 1e-16
    idx = (jax.lax.broadcasted_iota(jnp.int32, (PAD_R, PAD_C), 0) * PAD_C
           + jax.lax.broadcasted_iota(jnp.int32, (PAD_R, PAD_C), 1))
    s0 = jnp.tanh(d0_ref[...] / n0)
    mask1 = _select_topk(_ordered_key(s0), idx, K1)
    c1 = jnp.where(mask1, jnp.tanh(s0 * d1_ref[...] / n1), -2.0)
    mask2 = _select_topk(_ordered_key(c1), idx, K2)
    o_ref[...] = jnp.where(mask2, s0 * c1 * (1.0 / K2), 0.0)


def _wsum_body(c_ref, x_ref, o_ref):
    @pl.when(pl.program_id(0) == 0)
    def _():
        o_ref[...] = jnp.zeros_like(o_ref)
    o_ref[...] += jnp.dot(c_ref[0], x_ref[...],
                          preferred_element_type=jnp.float32)


@jax.jit
def kernel(x, edge_index, edge_attr, w0, w1):
    del edge_index, edge_attr
    W = jnp.stack([w0, w1], axis=1)

    d = pl.pallas_call(
        _matvec2_body,
        grid=(N_BLOCKS,),
        in_specs=[
            pl.BlockSpec((ROWS_BLK, IN_CH), lambda i: (i, 0)),
            pl.BlockSpec((IN_CH, 2), lambda i: (0, 0)),
        ],
        out_specs=pl.BlockSpec((ROWS_BLK, 2), lambda i: (i, 0)),
        out_shape=jax.ShapeDtypeStruct((N_NODES, 2), jnp.float32),
    )(x, W)

    pad = jnp.full((PAD_N - N_NODES,), -1e30, jnp.float32)
    d0 = jnp.concatenate([d[:, 0], pad]).reshape(PAD_R, PAD_C)
    d1 = jnp.concatenate([d[:, 1], jnp.zeros_like(pad)]).reshape(PAD_R, PAD_C)

    coeff = pl.pallas_call(
        _coeff_body,
        in_specs=[
            pl.BlockSpec((PAD_R, PAD_C), lambda: (0, 0)),
            pl.BlockSpec((PAD_R, PAD_C), lambda: (0, 0)),
            pl.BlockSpec((1, IN_CH), lambda: (0, 0)),
            pl.BlockSpec((1, IN_CH), lambda: (0, 0)),
        ],
        out_specs=pl.BlockSpec((PAD_R, PAD_C), lambda: (0, 0)),
        out_shape=jax.ShapeDtypeStruct((PAD_R, PAD_C), jnp.float32),
    )(d0, d1, w0.reshape(1, IN_CH), w1.reshape(1, IN_CH))

    c3 = coeff.reshape(PAD_N)[:N_NODES].reshape(N_BLOCKS, 1, ROWS_BLK)

    pooled = pl.pallas_call(
        _wsum_body,
        grid=(N_BLOCKS,),
        in_specs=[
            pl.BlockSpec((1, 1, ROWS_BLK), lambda i: (i, 0, 0)),
            pl.BlockSpec((ROWS_BLK, IN_CH), lambda i: (i, 0)),
        ],
        out_specs=pl.BlockSpec((1, IN_CH), lambda i: (0, 0)),
        out_shape=jax.ShapeDtypeStruct((1, IN_CH), jnp.float32),
    )(c3, x)

    return pooled


# speedup vs baseline: 3.4082x; 1.0094x over previous
import functools
import jax
import jax.numpy as jnp
import numpy as np
from jax.experimental import pallas as pl

N_NODES = 50000
IN_CH = 1443
K1 = 25000
K2 = 12500
ROWS_BLK = 2000
N_BLOCKS = N_NODES // ROWS_BLK
PAD_N = 50176
PAD_R, PAD_C = 392, 128


def _matvec2_body(x_ref, w_ref, o_ref):
    o_ref[...] = jnp.dot(x_ref[...], w_ref[...],
                         preferred_element_type=jnp.float32)


def _ordered_key(f):
    b = jax.lax.bitcast_convert_type(f, jnp.int32)
    return b ^ ((b >> 31) & jnp.int32(0x7FFFFFFF))


def _count_gt_threshold(key, k):
    def body(_, lohi):
        lo, hi = lohi
        mid = (lo & hi) + ((lo ^ hi) >> 1)
        cnt = jnp.sum((key > mid).astype(jnp.int32))
        big = cnt >= k
        return (jnp.where(big, mid + 1, lo), jnp.where(big, hi, mid))
    lo = jnp.int32(-2**31)
    hi = jnp.int32(2**31 - 1)
    lo, hi = jax.lax.fori_loop(0, 32, body, (lo, hi))
    return lo


def _select_topk(key, idx, k):
    t = _count_gt_threshold(key, k)
    above = key > t
    m = k - jnp.sum(above.astype(jnp.int32))
    tie = key == t

    def body(_, lohi):
        lo, hi = lohi
        mid = (lo + hi) >> 1
        cnt = jnp.sum((tie & (idx < mid)).astype(jnp.int32))
        big = cnt >= m
        return (jnp.where(big, lo, mid + 1), jnp.where(big, mid, hi))
    lo, hi = jax.lax.fori_loop(0, 17, body,
                               (jnp.int32(0), jnp.int32(PAD_N + 1)))
    return above | (tie & (idx < lo))


def _coeff_body(d0_ref, d1_ref, w0_ref, w1_ref, o_ref):
    n0 = jnp.sqrt(jnp.sum(w0_ref[...] * w0_ref[...])) + 1e-16
    n1 = jnp.sqrt(jnp.sum(w1_ref[...] * w1_ref[...])) + 1e-16
    idx = (jax.lax.broadcasted_iota(jnp.int32, (PAD_R, PAD_C), 0) * PAD_C
           + jax.lax.broadcasted_iota(jnp.int32, (PAD_R, PAD_C), 1))
    s0 = jnp.tanh(d0_ref[...] / n0)
    mask1 = _select_topk(_ordered_key(s0), idx, K1)
    c1 = jnp.where(mask1, jnp.tanh(s0 * d1_ref[...] / n1), -2.0)
    mask2 = _select_topk(_ordered_key(c1), idx, K2)
    o_ref[...] = jnp.where(mask2, s0 * c1 * (1.0 / K2), 0.0)


def _wsum_body(c_ref, x_ref, o_ref):
    @pl.when(pl.program_id(0) == 0)
    def _():
        o_ref[...] = jnp.zeros_like(o_ref)
    o_ref[...] += jnp.dot(c_ref[0], x_ref[...],
                          preferred_element_type=jnp.float32)


@jax.jit
def kernel(x, edge_index, edge_attr, w0, w1):
    del edge_index, edge_attr
    W = jnp.stack([w0, w1], axis=1)

    d = pl.pallas_call(
        _matvec2_body,
        grid=(N_BLOCKS,),
        in_specs=[
            pl.BlockSpec((ROWS_BLK, IN_CH), lambda i: (i, 0)),
            pl.BlockSpec((IN_CH, 2), lambda i: (0, 0)),
        ],
        out_specs=pl.BlockSpec((ROWS_BLK, 2), lambda i: (i, 0)),
        out_shape=jax.ShapeDtypeStruct((N_NODES, 2), jnp.float32),
    )(x, W)

    pad = jnp.full((PAD_N - N_NODES,), -1e30, jnp.float32)
    d0 = jnp.concatenate([d[:, 0], pad]).reshape(PAD_R, PAD_C)
    d1 = jnp.concatenate([d[:, 1], jnp.zeros_like(pad)]).reshape(PAD_R, PAD_C)

    coeff = pl.pallas_call(
        _coeff_body,
        in_specs=[
            pl.BlockSpec((PAD_R, PAD_C), lambda: (0, 0)),
            pl.BlockSpec((PAD_R, PAD_C), lambda: (0, 0)),
            pl.BlockSpec((1, IN_CH), lambda: (0, 0)),
            pl.BlockSpec((1, IN_CH), lambda: (0, 0)),
        ],
        out_specs=pl.BlockSpec((PAD_R, PAD_C), lambda: (0, 0)),
        out_shape=jax.ShapeDtypeStruct((PAD_R, PAD_C), jnp.float32),
    )(d0, d1, w0.reshape(1, IN_CH), w1.reshape(1, IN_CH))

    c3 = coeff.reshape(PAD_N)[:N_NODES].reshape(N_BLOCKS, 1, ROWS_BLK)

    pooled = pl.pallas_call(
        _wsum_body,
        grid=(N_BLOCKS,),
        in_specs=[
            pl.BlockSpec((1, 1, ROWS_BLK), lambda i: (i, 0, 0)),
            pl.BlockSpec((ROWS_BLK, IN_CH), lambda i: (i, 0)),
        ],
        out_specs=pl.BlockSpec((1, IN_CH), lambda i: (0, 0)),
        out_shape=jax.ShapeDtypeStruct((1, IN_CH), jnp.float32),
    )(c3, x)

    return pooled
